# baseline (speedup 1.0000x reference)
import jax
import jax.numpy as jnp
from jax import lax
from jax.experimental import pallas as pl
from jax.experimental.pallas import tpu as pltpu
from jax.experimental.pallas import tpu_sc as plsc

N_NODES = 10000
D = 128
E = 320000
LANES = 16
NC = 2
NS = 16
NW = NC * NS
N_PAD = 10240
EDGES_PER_W = E // NW
BLK = 80
NBLK = EDGES_PER_W // BLK
NRB = 3
NIB = 6
PRO = 5
ROWS_PER_TILE = N_PAD // NS

_GDN = lax.GatherDimensionNumbers(
    offset_dims=(), collapsed_slice_dims=(0,), start_index_map=(0,))


def _bcast_lane(v16, lane):
    idx = jnp.full((LANES, 1), lane, dtype=jnp.int32)
    return lax.gather(v16, idx, _GDN, (1,),
                      mode=lax.GatherScatterMode.PROMISE_IN_BOUNDS)


def _sc_body(x_hbm, ei_hbm, w_hbm, out_hbm, acc, src_all,
             dst_0, dst_1, dst_2, dst_3, dst_4, dst_5,
             w_0, w_1, w_2, w_3, w_4, w_5,
             buf_0, buf_1, buf_2,
             sg_0, sg_1, sg_2, si_0, si_1, si_2, si_3, si_4, si_5,
             ss_0, ss_1, ss_2):
    c = lax.axis_index("c")
    s = lax.axis_index("s")
    wid = s * NC + c
    ebase = wid * EDGES_PER_W

    dsts = (dst_0, dst_1, dst_2, dst_3, dst_4, dst_5)
    ws = (w_0, w_1, w_2, w_3, w_4, w_5)
    bufs = (buf_0, buf_1, buf_2)
    sgs = (sg_0, sg_1, sg_2)
    sis = (si_0, si_1, si_2, si_3, si_4, si_5)
    sss = (ss_0, ss_1, ss_2)

    pltpu.sync_copy(ei_hbm.at[pl.ds(ebase, EDGES_PER_W)], src_all)

    def zrow(i, carry):
        for j in range(D // LANES):
            buf_0[i, pl.ds(j * LANES, LANES)] = jnp.zeros((LANES,), jnp.float32)
        return carry

    lax.fori_loop(0, BLK, zrow, 0)
    for k in range(ROWS_PER_TILE // BLK):
        pltpu.sync_copy(buf_0, acc.at[pl.ds(s * ROWS_PER_TILE + k * BLK, BLK)])
    plsc.subcore_barrier()

    def issue_idx(i, u):
        pltpu.async_copy(ei_hbm.at[pl.ds(E + ebase + i * BLK, BLK)],
                         dsts[u], sis[u])
        pltpu.async_copy(w_hbm.at[pl.ds(ebase + i * BLK, BLK)],
                         ws[u], sis[u])

    def issue_gather(i, t):
        pltpu.async_copy(x_hbm.at[src_all.at[pl.ds(i * BLK, BLK)]],
                         bufs[t], sgs[t])

    def wait_in(i, t, u):
        pltpu.make_async_copy(
            x_hbm.at[src_all.at[pl.ds(i * BLK, BLK)]], bufs[t], sgs[t]).wait()
        pltpu.make_async_copy(
            ei_hbm.at[pl.ds(E + ebase + i * BLK, BLK)], dsts[u], sis[u]).wait()
        pltpu.make_async_copy(
            w_hbm.at[pl.ds(ebase + i * BLK, BLK)], ws[u], sis[u]).wait()

    def scale(t, u):
        buf, wref = bufs[t], ws[u]

        @plsc.parallel_loop(0, BLK // LANES)
        def grp(g):
            w16 = wref[pl.ds(g * LANES, LANES)]
            for l in range(LANES):
                bc = _bcast_lane(w16, l)
                e = g * LANES + l
                for j in range(D // LANES):
                    buf[e, pl.ds(j * LANES, LANES)] = (
                        buf[e, pl.ds(j * LANES, LANES)] * bc)

    def wait_sc(t, u):
        pltpu.make_async_copy(bufs[t], acc.at[dsts[u]], sss[t]).wait()

    def stage(j, t, u, first):
        wait_in(j, t, u)
        scale(t, u)
        pltpu.async_copy(bufs[t], acc.at[dsts[u]], sss[t], add=True)
        tp, up = (t + NRB - 1) % NRB, (u + NIB - 1) % NIB
        if not first:
            wait_sc(tp, up)
        if isinstance(j, int):
            if j + 2 < NBLK:
                issue_gather(j + 2, tp)
            if j + 5 < NBLK:
                issue_idx(j + 5, up)
        else:

            @pl.when(j + 2 < NBLK)
            def _():
                issue_gather(j + 2, tp)

            @pl.when(j + 5 < NBLK)
            def _():
                issue_idx(j + 5, up)

    for i in range(PRO):
        issue_idx(i, i)
    issue_gather(0, 0)
    issue_gather(1, 1)
    stage(0, 0, 0, True)
    for i in range(1, PRO):
        stage(i, i % NRB, i, False)

    def six(k, carry):
        j0 = PRO + NIB * k
        for m in range(NIB):
            jm = PRO + m
            stage(j0 + m, jm % NRB, jm % NIB, False)
        return carry

    lax.fori_loop(0, (NBLK - PRO) // NIB, six, 0)
    wait_sc((NBLK - 1) % NRB, (NBLK - 1) % NIB)
    plsc.subcore_barrier()

    pltpu.sync_copy(acc.at[pl.ds(s * ROWS_PER_TILE, ROWS_PER_TILE)],
                    out_hbm.at[c, pl.ds(s * ROWS_PER_TILE, ROWS_PER_TILE)])


_sc_call = pl.kernel(
    _sc_body,
    out_type=jax.ShapeDtypeStruct((NC, N_PAD, D), jnp.float32),
    mesh=plsc.VectorSubcoreMesh(core_axis_name="c", subcore_axis_name="s"),
    scratch_types=(
        [pltpu.VMEM_SHARED((N_PAD, D), jnp.float32)]
        + [pltpu.VMEM((EDGES_PER_W,), jnp.int32)]
        + [pltpu.VMEM((BLK,), jnp.int32) for _ in range(NIB)]
        + [pltpu.VMEM((BLK,), jnp.float32) for _ in range(NIB)]
        + [pltpu.VMEM((BLK, D), jnp.float32) for _ in range(NRB)]
        + [pltpu.SemaphoreType.DMA for _ in range(NRB + NIB + NRB)]
    ),
    name="gcn_spmm_sc",
)

_CBLK = 10000


def _combine_body(p_ref, q_ref, o_ref):
    o_ref[...] = p_ref[0] + q_ref[0]


_combine = pl.pallas_call(
    _combine_body,
    grid=(N_NODES // _CBLK,),
    in_specs=[
        pl.BlockSpec((1, _CBLK, D), lambda i: (0, i, 0)),
        pl.BlockSpec((1, _CBLK, D), lambda i: (1, i, 0)),
    ],
    out_specs=pl.BlockSpec((_CBLK, D), lambda i: (i, 0)),
    out_shape=jax.ShapeDtypeStruct((N_NODES, D), jnp.float32),
)


def kernel(input, edge_index, edge_weight):
    partials = _sc_call(input, edge_index.reshape(2 * E), edge_weight)
    return _combine(partials, partials)


# speedup vs baseline: 1.2996x; 1.2996x over previous
import jax
import jax.numpy as jnp
from jax import lax
from jax.experimental import pallas as pl
from jax.experimental.pallas import tpu as pltpu
from jax.experimental.pallas import tpu_sc as plsc

N_NODES = 10000
D = 128
E = 320000
LANES = 16
NC = 2
NS = 16
NW = NC * NS
N_PAD = 10240
EDGES_PER_W = E // NW
BLK = 80
NBLK = EDGES_PER_W // BLK
NRB = 3
NIB = 6
PRO = 5
ROWS_PER_TILE = N_PAD // NS

_GDN = lax.GatherDimensionNumbers(
    offset_dims=(), collapsed_slice_dims=(0,), start_index_map=(0,))


def _bcast_lane(v16, lane):
    idx = jnp.full((LANES, 1), lane, dtype=jnp.int32)
    return lax.gather(v16, idx, _GDN, (1,),
                      mode=lax.GatherScatterMode.PROMISE_IN_BOUNDS)


def _sc_body(x_hbm, ei_hbm, w_hbm, out_hbm, acc, src_all,
             dst_0, dst_1, dst_2, dst_3, dst_4, dst_5,
             w_0, w_1, w_2, w_3, w_4, w_5,
             buf_0, buf_1, buf_2,
             sg_0, sg_1, sg_2, si_0, si_1, si_2, si_3, si_4, si_5,
             ss_0, ss_1, ss_2):
    c = lax.axis_index("c")
    s = lax.axis_index("s")
    wid = s * NC + c
    ebase = wid * EDGES_PER_W

    dsts = (dst_0, dst_1, dst_2, dst_3, dst_4, dst_5)
    ws = (w_0, w_1, w_2, w_3, w_4, w_5)
    bufs = (buf_0, buf_1, buf_2)
    sgs = (sg_0, sg_1, sg_2)
    sis = (si_0, si_1, si_2, si_3, si_4, si_5)
    sss = (ss_0, ss_1, ss_2)

    pltpu.sync_copy(ei_hbm.at[pl.ds(ebase, EDGES_PER_W)], src_all)

    def zrow(i, carry):
        for j in range(D // LANES):
            buf_0[i, pl.ds(j * LANES, LANES)] = jnp.zeros((LANES,), jnp.float32)
        return carry

    lax.fori_loop(0, BLK, zrow, 0)
    for k in range(ROWS_PER_TILE // BLK):
        pltpu.sync_copy(buf_0, acc.at[pl.ds(s * ROWS_PER_TILE + k * BLK, BLK)])
    plsc.subcore_barrier()

    def issue_idx(i, u):
        pltpu.async_copy(ei_hbm.at[pl.ds(E + ebase + i * BLK, BLK)],
                         dsts[u], sis[u])
        pltpu.async_copy(w_hbm.at[pl.ds(ebase + i * BLK, BLK)],
                         ws[u], sis[u])

    def issue_gather(i, t):
        pltpu.async_copy(x_hbm.at[src_all.at[pl.ds(i * BLK, BLK)]],
                         bufs[t], sgs[t])

    def wait_in(i, t, u):
        pltpu.make_async_copy(
            x_hbm.at[src_all.at[pl.ds(i * BLK, BLK)]], bufs[t], sgs[t]).wait()
        pltpu.make_async_copy(
            ei_hbm.at[pl.ds(E + ebase + i * BLK, BLK)], dsts[u], sis[u]).wait()
        pltpu.make_async_copy(
            w_hbm.at[pl.ds(ebase + i * BLK, BLK)], ws[u], sis[u]).wait()

    def scale(t, u):
        buf, wref = bufs[t], ws[u]

        def grp(g, gcarry):
            w16 = wref[pl.ds(g * LANES, LANES)]
            for l in range(LANES):
                bc = _bcast_lane(w16, l)
                e = g * LANES + l
                for j in range(D // LANES):
                    buf[e, pl.ds(j * LANES, LANES)] = (
                        buf[e, pl.ds(j * LANES, LANES)] * bc)
            return gcarry

        lax.fori_loop(0, BLK // LANES, grp, 0)

    def wait_sc(t, u):
        pltpu.make_async_copy(bufs[t], acc.at[dsts[u]], sss[t]).wait()

    def stage(j, t, u, first):
        wait_in(j, t, u)
        scale(t, u)
        pltpu.async_copy(bufs[t], acc.at[dsts[u]], sss[t], add=True)
        tp, up = (t + NRB - 1) % NRB, (u + NIB - 1) % NIB
        if not first:
            wait_sc(tp, up)
        if isinstance(j, int):
            if j + 2 < NBLK:
                issue_gather(j + 2, tp)
            if j + 5 < NBLK:
                issue_idx(j + 5, up)
        else:

            @pl.when(j + 2 < NBLK)
            def _():
                issue_gather(j + 2, tp)

            @pl.when(j + 5 < NBLK)
            def _():
                issue_idx(j + 5, up)

    for i in range(PRO):
        issue_idx(i, i)
    issue_gather(0, 0)
    issue_gather(1, 1)
    stage(0, 0, 0, True)
    for i in range(1, PRO):
        stage(i, i % NRB, i, False)

    def six(k, carry):
        j0 = PRO + NIB * k
        for m in range(NIB):
            jm = PRO + m
            stage(j0 + m, jm % NRB, jm % NIB, False)
        return carry

    lax.fori_loop(0, (NBLK - PRO) // NIB, six, 0)
    wait_sc((NBLK - 1) % NRB, (NBLK - 1) % NIB)
    plsc.subcore_barrier()

    pltpu.sync_copy(acc.at[pl.ds(s * ROWS_PER_TILE, ROWS_PER_TILE)],
                    out_hbm.at[c, pl.ds(s * ROWS_PER_TILE, ROWS_PER_TILE)])


_sc_call = pl.kernel(
    _sc_body,
    out_type=jax.ShapeDtypeStruct((NC, N_PAD, D), jnp.float32),
    mesh=plsc.VectorSubcoreMesh(core_axis_name="c", subcore_axis_name="s"),
    scratch_types=(
        [pltpu.VMEM_SHARED((N_PAD, D), jnp.float32)]
        + [pltpu.VMEM((EDGES_PER_W,), jnp.int32)]
        + [pltpu.VMEM((BLK,), jnp.int32) for _ in range(NIB)]
        + [pltpu.VMEM((BLK,), jnp.float32) for _ in range(NIB)]
        + [pltpu.VMEM((BLK, D), jnp.float32) for _ in range(NRB)]
        + [pltpu.SemaphoreType.DMA for _ in range(NRB + NIB + NRB)]
    ),
    name="gcn_spmm_sc",
)

_CBLK = 10000


def _combine_body(p_ref, q_ref, o_ref):
    o_ref[...] = p_ref[0] + q_ref[0]


_combine = pl.pallas_call(
    _combine_body,
    grid=(N_NODES // _CBLK,),
    in_specs=[
        pl.BlockSpec((1, _CBLK, D), lambda i: (0, i, 0)),
        pl.BlockSpec((1, _CBLK, D), lambda i: (1, i, 0)),
    ],
    out_specs=pl.BlockSpec((_CBLK, D), lambda i: (i, 0)),
    out_shape=jax.ShapeDtypeStruct((N_NODES, D), jnp.float32),
)


def kernel(input, edge_index, edge_weight):
    partials = _sc_call(input, edge_index.reshape(2 * E), edge_weight)
    return _combine(partials, partials)
